# baseline (speedup 1.0000x reference)
import functools

import numpy as np

import jax
import jax.numpy as jnp
from jax import lax
from jax.experimental import pallas as pl
from jax.experimental.pallas import tpu as pltpu
from jax.experimental.pallas import tpu_sc as plsc

B = 16384
F = 26
FP = 32
E = 16
NC = 2
NS = 16
NW = NC * NS
BPW = B // NW
CHUNK = 64
RPC = CHUNK * FP

RL_CH = 16384
RL_GRID = 62
RL_SLAB = RL_CH // 8
RL_SHIFT = RL_SLAB.bit_length() - 1

PK_BLK = 2048
PK_GRID = B // PK_BLK
PK_SLAB = PK_BLK // 4


def _relayout_body(in_ref, q_ref, out_ref):
    x = in_ref[...]
    ch = x.shape[1]
    x2 = x.reshape(128, ch // 8)
    out_ref[...] = lax.dot_general(
        x2, q_ref[...], (((0,), (0,)), ((), ())),
        preferred_element_type=jnp.float32)


def _perm_q():
    q = np.zeros((128, 128), np.float32)
    for d in range(E):
        for j in range(8):
            q[8 * d + j, E * j + d] = 1.0
    return jnp.asarray(q)


def _linearize_table_tc(emb_table):
    tt = jnp.transpose(emb_table)
    lin = pl.pallas_call(
        _relayout_body,
        grid=(RL_GRID,),
        in_specs=[pl.BlockSpec((E, RL_CH), lambda i: (0, i)),
                  pl.BlockSpec((128, 128), lambda i: (0, 0))],
        out_specs=pl.BlockSpec((RL_CH // 8, 128), lambda i: (i, 0)),
        out_shape=jax.ShapeDtypeStruct((RL_GRID * RL_CH // 8, 128),
                                       jnp.float32),
    )(tt, _perm_q())
    return lin.reshape(RL_GRID * RL_CH, E)


def _pack_body(idx_ref, val_ref, p_ref, iout_ref, vout_ref):
    xi = idx_ref[...]
    vidx = ((xi & ~(RL_CH - 1)) | ((xi & (RL_SLAB - 1)) << 3)
            | ((xi >> RL_SHIFT) & 7)).astype(jnp.float32)
    xv = val_ref[...]
    iacc = None
    vacc = None
    for k in range(4):
        p_k = p_ref[k]
        sl = slice(PK_SLAB * k, PK_SLAB * (k + 1))
        it = jnp.dot(vidx[sl], p_k, preferred_element_type=jnp.float32)
        vt = jnp.dot(xv[sl], p_k, preferred_element_type=jnp.float32)
        iacc = it if iacc is None else iacc + it
        vacc = vt if vacc is None else vacc + vt
    iout_ref[...] = iacc.astype(jnp.int32)
    vout_ref[...] = vacc


def _pack_p():
    p = np.zeros((4, F, 128), np.float32)
    for k in range(4):
        for g in range(F):
            p[k, g, FP * k + g] = 1.0
    return jnp.asarray(p)


def _pack_inputs_tc(feat_index, feat_value):
    return pl.pallas_call(
        _pack_body,
        grid=(PK_GRID,),
        in_specs=[pl.BlockSpec((PK_BLK, F), lambda i: (i, 0)),
                  pl.BlockSpec((PK_BLK, F), lambda i: (i, 0)),
                  pl.BlockSpec((4, F, 128), lambda i: (0, 0, 0))],
        out_specs=[pl.BlockSpec((PK_SLAB, 128), lambda i: (i, 0)),
                   pl.BlockSpec((PK_SLAB, 128), lambda i: (i, 0))],
        out_shape=[jax.ShapeDtypeStruct((B // 4, 128), jnp.int32),
                   jax.ShapeDtypeStruct((B // 4, 128), jnp.float32)],
    )(feat_index.astype(jnp.int32), feat_value, _pack_p())


def _sc_body(table, idx_hbm, val_hbm, out_hbm,
             idx_v, val_v, rows_a, rows_b, out_v, sem_a, sem_b):
    c = lax.axis_index("c")
    s = lax.axis_index("s")
    wid = s * NC + c

    pltpu.sync_copy(idx_hbm.at[pl.ds(wid * 128, 128), :], idx_v)
    pltpu.sync_copy(val_hbm.at[pl.ds(wid * 128, 128), :], val_v)

    bufs = (rows_a, rows_b)
    sems = (sem_a, sem_b)

    def gather_desc(k, rbase, slot, i):
        return pltpu.make_async_copy(
            table.at[idx_v.at[rbase + i, pl.ds(FP * k, FP)]],
            bufs[slot].at[pl.ds(i * FP, FP), :],
            sems[slot])

    def fire(k, rbase, slot):
        def body(i, _):
            gather_desc(k, rbase, slot, i).start()
            return ()
        lax.fori_loop(0, CHUNK, body, ())

    def drain(k, rbase, slot):
        def body(i, _):
            gather_desc(k, rbase, slot, i).wait()
            return ()
        lax.fori_loop(0, CHUNK, body, ())

    def compute(k, rbase, slot):
        rows = bufs[slot]

        def body(i, _):
            r = rbase + i
            r0 = i * FP
            v_lo = val_v[r, pl.ds(FP * k, E)]
            v_hi = val_v[r, pl.ds(FP * k + E, E)]
            acc0 = jnp.zeros((E,), jnp.float32)
            acc1 = jnp.zeros((E,), jnp.float32)
            sq0 = jnp.zeros((E,), jnp.float32)
            sq1 = jnp.zeros((E,), jnp.float32)
            for f in range(F):
                row = rows[r0 + f, :]
                scalar = v_lo[f] if f < E else v_hi[f - E]
                wv = row * jnp.broadcast_to(scalar, (E,))
                if f % 2 == 0:
                    acc0 = acc0 + wv
                    sq0 = sq0 + wv * wv
                else:
                    acc1 = acc1 + wv
                    sq1 = sq1 + wv * wv
            acc = acc0 + acc1
            sq = sq0 + sq1
            out_v[128 * k + r, :] = (acc * acc - sq) * 0.5
            return ()

        lax.fori_loop(0, CHUNK, body, ())

    chunks = [(k, rbase) for k in range(4) for rbase in (0, CHUNK)]
    fire(*chunks[0], 0)
    for ci, (k, rbase) in enumerate(chunks):
        slot = ci % 2
        drain(k, rbase, slot)
        if ci + 1 < len(chunks):
            fire(*chunks[ci + 1], 1 - slot)
        compute(k, rbase, slot)

    bq = (wid // 4) * 2048 + (wid % 4) * 128
    for k in range(4):
        pltpu.sync_copy(out_v.at[pl.ds(128 * k, 128), :],
                        out_hbm.at[pl.ds(bq + 512 * k, 128), :])


def _bi_interaction_sc(idx_packed, val_packed, table_lin):
    mesh = plsc.VectorSubcoreMesh(core_axis_name="c", subcore_axis_name="s")
    fn = pl.kernel(
        _sc_body,
        out_type=jax.ShapeDtypeStruct((B, E), jnp.float32),
        mesh=mesh,
        compiler_params=pltpu.CompilerParams(
            use_tc_tiling_on_sc=False, needs_layout_passes=False),
        scratch_types=[
            pltpu.VMEM((128, 128), jnp.int32),
            pltpu.VMEM((128, 128), jnp.float32),
            pltpu.VMEM((RPC, E), jnp.float32),
            pltpu.VMEM((RPC, E), jnp.float32),
            pltpu.VMEM((BPW, E), jnp.float32),
            pltpu.SemaphoreType.DMA,
            pltpu.SemaphoreType.DMA,
        ],
    )
    return fn(table_lin, idx_packed, val_packed)


def _mlp_body(bi_ref, w1_ref, b1_ref, w2_ref, b2_ref, wo_ref, bo_ref, out_ref):
    x = bi_ref[...]
    h = jnp.dot(x, w1_ref[...], preferred_element_type=jnp.float32)
    h = jnp.maximum(h + b1_ref[...], 0.0)
    h = jnp.dot(h, w2_ref[...], preferred_element_type=jnp.float32)
    h = jnp.maximum(h + b2_ref[...], 0.0)
    o = jnp.sum(h * wo_ref[...], axis=1, keepdims=True) + bo_ref[...]
    out_ref[...] = 1.0 / (1.0 + jnp.exp(-o))


def _mlp_tc(bi, W1, b1, W2, b2, Wo, bo):
    nblk = 8
    blk = B // nblk
    return pl.pallas_call(
        _mlp_body,
        grid=(nblk,),
        in_specs=[
            pl.BlockSpec((blk, E), lambda i: (i, 0)),
            pl.BlockSpec((E, 32), lambda i: (0, 0)),
            pl.BlockSpec((1, 32), lambda i: (0, 0)),
            pl.BlockSpec((32, 32), lambda i: (0, 0)),
            pl.BlockSpec((1, 32), lambda i: (0, 0)),
            pl.BlockSpec((1, 32), lambda i: (0, 0)),
            pl.BlockSpec((1, 1), lambda i: (0, 0)),
        ],
        out_specs=pl.BlockSpec((blk, 1), lambda i: (i, 0)),
        out_shape=jax.ShapeDtypeStruct((B, 1), jnp.float32),
    )(bi, W1, b1.reshape(1, 32), W2, b2.reshape(1, 32),
      Wo.reshape(1, 32), bo.reshape(1, 1))


@jax.jit
def _nfm(feat_index, feat_value, emb_table, W1, b1, W2, b2, Wo, bo):
    table_lin = _linearize_table_tc(emb_table)
    idx_packed, val_packed = _pack_inputs_tc(feat_index, feat_value)
    bi = _bi_interaction_sc(idx_packed, val_packed, table_lin)
    return _mlp_tc(bi, W1, b1, W2, b2, Wo, bo)


def kernel(feat_index, feat_value, emb_table, W1, b1, W2, b2, Wo, bo):
    return _nfm(feat_index, feat_value, emb_table, W1, b1, W2, b2, Wo, bo)


# speedup vs baseline: 3.3192x; 3.3192x over previous
import functools

import numpy as np

import jax
import jax.numpy as jnp
from jax import lax
from jax.experimental import pallas as pl
from jax.experimental.pallas import tpu as pltpu
from jax.experimental.pallas import tpu_sc as plsc

B = 16384
F = 26
FP = 32
E = 16
NC = 2
NS = 16
NW = NC * NS
BPW = B // NW
CHUNK = 64
RPC = CHUNK * FP

RL_CH = 16384
RL_GRID = 62
RL_SLAB = RL_CH // 8
RL_SHIFT = RL_SLAB.bit_length() - 1

PK_BLK = 2048
PK_GRID = B // PK_BLK
PK_SLAB = PK_BLK // 4


def _relayout_body(in_ref, q_ref, out_ref):
    x = in_ref[...]
    ch = x.shape[1]
    x2 = x.reshape(128, ch // 8)
    out_ref[...] = lax.dot_general(
        x2, q_ref[...], (((0,), (0,)), ((), ())),
        preferred_element_type=jnp.float32)


def _perm_q():
    q = np.zeros((128, 128), np.float32)
    for d in range(E):
        for j in range(8):
            q[8 * d + j, E * j + d] = 1.0
    return jnp.asarray(q)


def _linearize_table_tc(emb_table):
    tt = jnp.transpose(emb_table)
    lin = pl.pallas_call(
        _relayout_body,
        grid=(RL_GRID,),
        in_specs=[pl.BlockSpec((E, RL_CH), lambda i: (0, i)),
                  pl.BlockSpec((128, 128), lambda i: (0, 0))],
        out_specs=pl.BlockSpec((RL_CH // 8, 128), lambda i: (i, 0)),
        out_shape=jax.ShapeDtypeStruct((RL_GRID * RL_CH // 8, 128),
                                       jnp.float32),
    )(tt, _perm_q())
    return lin.reshape(RL_GRID * RL_CH, E)


def _pack_body(idx_ref, val_ref, p_ref, iout_ref, vout_ref):
    xi = idx_ref[...]
    vidx = ((xi & ~(RL_CH - 1)) | ((xi & (RL_SLAB - 1)) << 3)
            | ((xi >> RL_SHIFT) & 7)).astype(jnp.float32)
    xv = val_ref[...]
    iacc = None
    vacc = None
    for k in range(4):
        p_k = p_ref[k]
        sl = slice(PK_SLAB * k, PK_SLAB * (k + 1))
        it = jnp.dot(vidx[sl], p_k, preferred_element_type=jnp.float32)
        vt = jnp.dot(xv[sl], p_k, preferred_element_type=jnp.float32)
        iacc = it if iacc is None else iacc + it
        vacc = vt if vacc is None else vacc + vt
    lane = lax.broadcasted_iota(jnp.int32, (PK_SLAB, 128), 1)
    rowi = lax.broadcasted_iota(jnp.int32, (PK_SLAB, 128), 0)
    pad = (lane % FP) >= F
    iout_ref[...] = jnp.where(pad, (rowi * 8 + (lane & 7)) & (RL_CH - 1),
                              iacc.astype(jnp.int32))
    vout_ref[...] = vacc


def _pack_p():
    p = np.zeros((4, F, 128), np.float32)
    for k in range(4):
        for g in range(F):
            p[k, g, FP * k + g] = 1.0
    return jnp.asarray(p)


def _pack_inputs_tc(feat_index, feat_value):
    return pl.pallas_call(
        _pack_body,
        grid=(PK_GRID,),
        in_specs=[pl.BlockSpec((PK_BLK, F), lambda i: (i, 0)),
                  pl.BlockSpec((PK_BLK, F), lambda i: (i, 0)),
                  pl.BlockSpec((4, F, 128), lambda i: (0, 0, 0))],
        out_specs=[pl.BlockSpec((PK_SLAB, 128), lambda i: (i, 0)),
                   pl.BlockSpec((PK_SLAB, 128), lambda i: (i, 0))],
        out_shape=[jax.ShapeDtypeStruct((B // 4, 128), jnp.int32),
                   jax.ShapeDtypeStruct((B // 4, 128), jnp.float32)],
    )(feat_index.astype(jnp.int32), feat_value, _pack_p())


def _sc_body(table, idx_hbm, val_hbm, out_hbm,
             idx_v, val_v, rows_a, rows_b, out_v, sem_a, sem_b):
    c = lax.axis_index("c")
    s = lax.axis_index("s")
    wid = s * NC + c

    pltpu.sync_copy(idx_hbm.at[pl.ds(wid * BPW, BPW), :], idx_v)
    pltpu.sync_copy(val_hbm.at[pl.ds(wid * BPW, BPW), :], val_v)

    bufs = (rows_a, rows_b)
    sems = (sem_a, sem_b)

    def gather_desc(jbase, slot, i):
        return pltpu.make_async_copy(
            table.at[idx_v.at[jbase + i]],
            bufs[slot].at[pl.ds(i * FP, FP), :],
            sems[slot])

    def fire(jbase, slot):
        def body(i, _):
            gather_desc(jbase, slot, i).start()
            return ()
        lax.fori_loop(0, CHUNK, body, ())

    def drain(jbase, slot):
        def body(i, _):
            gather_desc(jbase, slot, i).wait()
            return ()
        lax.fori_loop(0, CHUNK, body, ())

    def compute(jbase, slot):
        rows = bufs[slot]

        def body(i, _):
            j = jbase + i
            r0 = i * FP
            v_lo = val_v[j, pl.ds(0, E)]
            v_hi = val_v[j, pl.ds(E, E)]
            acc0 = jnp.zeros((E,), jnp.float32)
            acc1 = jnp.zeros((E,), jnp.float32)
            sq0 = jnp.zeros((E,), jnp.float32)
            sq1 = jnp.zeros((E,), jnp.float32)
            for f in range(F):
                row = rows[r0 + f, :]
                scalar = v_lo[f] if f < E else v_hi[f - E]
                wv = row * jnp.broadcast_to(scalar, (E,))
                if f % 2 == 0:
                    acc0 = acc0 + wv
                    sq0 = sq0 + wv * wv
                else:
                    acc1 = acc1 + wv
                    sq1 = sq1 + wv * wv
            acc = acc0 + acc1
            sq = sq0 + sq1
            out_v[128 * (j & 3) + (j >> 2), :] = (acc * acc - sq) * 0.5
            return ()

        lax.fori_loop(0, CHUNK, body, ())

    nchunk = BPW // CHUNK
    fire(0, 0)
    for ci in range(nchunk):
        slot = ci % 2
        drain(ci * CHUNK, slot)
        if ci + 1 < nchunk:
            fire((ci + 1) * CHUNK, 1 - slot)
        compute(ci * CHUNK, slot)

    bq = (wid // 4) * 2048 + (wid % 4) * 128
    for k in range(4):
        pltpu.sync_copy(out_v.at[pl.ds(128 * k, 128), :],
                        out_hbm.at[pl.ds(bq + 512 * k, 128), :])


def _bi_interaction_sc(idx_packed, val_packed, table_lin):
    mesh = plsc.VectorSubcoreMesh(core_axis_name="c", subcore_axis_name="s")
    fn = pl.kernel(
        _sc_body,
        out_type=jax.ShapeDtypeStruct((B, E), jnp.float32),
        mesh=mesh,
        compiler_params=pltpu.CompilerParams(
            use_tc_tiling_on_sc=False, needs_layout_passes=False),
        scratch_types=[
            pltpu.VMEM((BPW, FP), jnp.int32),
            pltpu.VMEM((BPW, FP), jnp.float32),
            pltpu.VMEM((RPC, E), jnp.float32),
            pltpu.VMEM((RPC, E), jnp.float32),
            pltpu.VMEM((BPW, E), jnp.float32),
            pltpu.SemaphoreType.DMA,
            pltpu.SemaphoreType.DMA,
        ],
    )
    return fn(table_lin, idx_packed, val_packed)


def _mlp_body(bi_ref, w1_ref, b1_ref, w2_ref, b2_ref, wo_ref, bo_ref, out_ref):
    x = bi_ref[...]
    h = jnp.dot(x, w1_ref[...], preferred_element_type=jnp.float32)
    h = jnp.maximum(h + b1_ref[...], 0.0)
    h = jnp.dot(h, w2_ref[...], preferred_element_type=jnp.float32)
    h = jnp.maximum(h + b2_ref[...], 0.0)
    o = jnp.sum(h * wo_ref[...], axis=1, keepdims=True) + bo_ref[...]
    out_ref[...] = 1.0 / (1.0 + jnp.exp(-o))


def _mlp_tc(bi, W1, b1, W2, b2, Wo, bo):
    nblk = 8
    blk = B // nblk
    return pl.pallas_call(
        _mlp_body,
        grid=(nblk,),
        in_specs=[
            pl.BlockSpec((blk, E), lambda i: (i, 0)),
            pl.BlockSpec((E, 32), lambda i: (0, 0)),
            pl.BlockSpec((1, 32), lambda i: (0, 0)),
            pl.BlockSpec((32, 32), lambda i: (0, 0)),
            pl.BlockSpec((1, 32), lambda i: (0, 0)),
            pl.BlockSpec((1, 32), lambda i: (0, 0)),
            pl.BlockSpec((1, 1), lambda i: (0, 0)),
        ],
        out_specs=pl.BlockSpec((blk, 1), lambda i: (i, 0)),
        out_shape=jax.ShapeDtypeStruct((B, 1), jnp.float32),
    )(bi, W1, b1.reshape(1, 32), W2, b2.reshape(1, 32),
      Wo.reshape(1, 32), bo.reshape(1, 1))


@jax.jit
def _nfm(feat_index, feat_value, emb_table, W1, b1, W2, b2, Wo, bo):
    table_lin = _linearize_table_tc(emb_table)
    idx_packed, val_packed = _pack_inputs_tc(feat_index, feat_value)
    bi = _bi_interaction_sc(idx_packed.reshape(B, FP),
                            val_packed.reshape(B, FP), table_lin)
    return _mlp_tc(bi, W1, b1, W2, b2, Wo, bo)


def kernel(feat_index, feat_value, emb_table, W1, b1, W2, b2, Wo, bo):
    return _nfm(feat_index, feat_value, emb_table, W1, b1, W2, b2, Wo, bo)


# speedup vs baseline: 3.8571x; 1.1621x over previous
import functools

import numpy as np

import jax
import jax.numpy as jnp
from jax import lax
from jax.experimental import pallas as pl
from jax.experimental.pallas import tpu as pltpu
from jax.experimental.pallas import tpu_sc as plsc

B = 16384
F = 26
FP = 32
E = 16
NC = 2
NS = 16
NW = NC * NS
BPW = B // NW
CHUNK = 64
RPC = CHUNK * FP

RL_CH = 16384
RL_GRID = 62
RL_SLAB = RL_CH // 8
RL_SHIFT = RL_SLAB.bit_length() - 1

PK_BLK = 2048
PK_GRID = B // PK_BLK
PK_SLAB = PK_BLK // 4


def _relayout_body(in_ref, q_ref, out_ref):
    x = in_ref[...]
    ch = x.shape[1]
    x2 = x.reshape(128, ch // 8)
    out_ref[...] = lax.dot_general(
        x2, q_ref[...], (((0,), (0,)), ((), ())),
        preferred_element_type=jnp.float32)


def _perm_q():
    q = np.zeros((128, 128), np.float32)
    for d in range(E):
        for j in range(8):
            q[8 * d + j, E * j + d] = 1.0
    return jnp.asarray(q)


def _linearize_table_tc(emb_table):
    tt = jnp.transpose(emb_table)
    lin = pl.pallas_call(
        _relayout_body,
        grid=(RL_GRID,),
        in_specs=[pl.BlockSpec((E, RL_CH), lambda i: (0, i)),
                  pl.BlockSpec((128, 128), lambda i: (0, 0))],
        out_specs=pl.BlockSpec((RL_CH // 8, 128), lambda i: (i, 0)),
        out_shape=jax.ShapeDtypeStruct((RL_GRID * RL_CH // 8, 128),
                                       jnp.float32),
    )(tt, _perm_q())
    return lin.reshape(RL_GRID * RL_CH, E)


def _pack_body(idx_ref, val_ref, p_ref, iout_ref, vout_ref):
    xi = idx_ref[...]
    vidx = ((xi & ~(RL_CH - 1)) | ((xi & (RL_SLAB - 1)) << 3)
            | ((xi >> RL_SHIFT) & 7)).astype(jnp.float32)
    xv = val_ref[...]
    iacc = None
    vacc = None
    for k in range(4):
        p_k = p_ref[k]
        sl = slice(PK_SLAB * k, PK_SLAB * (k + 1))
        it = lax.dot_general(vidx[:, sl], p_k, (((0,), (0,)), ((), ())),
                             preferred_element_type=jnp.float32)
        vt = lax.dot_general(xv[:, sl], p_k, (((0,), (0,)), ((), ())),
                             preferred_element_type=jnp.float32)
        iacc = it if iacc is None else iacc + it
        vacc = vt if vacc is None else vacc + vt
    lane = lax.broadcasted_iota(jnp.int32, (PK_SLAB, 128), 1)
    rowi = lax.broadcasted_iota(jnp.int32, (PK_SLAB, 128), 0)
    pad = (lane % FP) >= F
    iout_ref[...] = jnp.where(pad, (rowi * 8 + (lane & 7)) & (RL_CH - 1),
                              iacc.astype(jnp.int32))
    vout_ref[...] = vacc


def _pack_p():
    p = np.zeros((4, F, 128), np.float32)
    for k in range(4):
        for g in range(F):
            p[k, g, FP * k + g] = 1.0
    return jnp.asarray(p)


def _pack_inputs_tc(feat_index, feat_value):
    return pl.pallas_call(
        _pack_body,
        grid=(PK_GRID,),
        in_specs=[pl.BlockSpec((F, PK_BLK), lambda i: (0, i)),
                  pl.BlockSpec((F, PK_BLK), lambda i: (0, i)),
                  pl.BlockSpec((4, F, 128), lambda i: (0, 0, 0))],
        out_specs=[pl.BlockSpec((PK_SLAB, 128), lambda i: (i, 0)),
                   pl.BlockSpec((PK_SLAB, 128), lambda i: (i, 0))],
        out_shape=[jax.ShapeDtypeStruct((B // 4, 128), jnp.int32),
                   jax.ShapeDtypeStruct((B // 4, 128), jnp.float32)],
    )(jnp.transpose(feat_index.astype(jnp.int32)),
      jnp.transpose(feat_value), _pack_p())


def _sc_body(table, idx_hbm, val_hbm, out_hbm,
             idx_v, val_v, rows_a, rows_b, out_v, sem_a, sem_b):
    c = lax.axis_index("c")
    s = lax.axis_index("s")
    wid = s * NC + c

    pltpu.sync_copy(idx_hbm.at[pl.ds(wid * BPW, BPW), :], idx_v)
    pltpu.sync_copy(val_hbm.at[pl.ds(wid * BPW, BPW), :], val_v)

    bufs = (rows_a, rows_b)
    sems = (sem_a, sem_b)

    def gather_desc(jbase, slot, i):
        return pltpu.make_async_copy(
            table.at[idx_v.at[jbase + i]],
            bufs[slot].at[pl.ds(i * FP, FP), :],
            sems[slot])

    def fire(jbase, slot):
        def body(i, _):
            gather_desc(jbase, slot, i).start()
            return ()
        lax.fori_loop(0, CHUNK, body, ())

    def drain(jbase, slot):
        def body(i, _):
            gather_desc(jbase, slot, i).wait()
            return ()
        lax.fori_loop(0, CHUNK, body, ())

    def compute(jbase, slot):
        rows = bufs[slot]

        def body(i, _):
            j = jbase + i
            r0 = i * FP
            v_lo = val_v[j, pl.ds(0, E)]
            v_hi = val_v[j, pl.ds(E, E)]
            acc0 = jnp.zeros((E,), jnp.float32)
            acc1 = jnp.zeros((E,), jnp.float32)
            sq0 = jnp.zeros((E,), jnp.float32)
            sq1 = jnp.zeros((E,), jnp.float32)
            for f in range(F):
                row = rows[r0 + f, :]
                scalar = v_lo[f] if f < E else v_hi[f - E]
                wv = row * jnp.broadcast_to(scalar, (E,))
                if f % 2 == 0:
                    acc0 = acc0 + wv
                    sq0 = sq0 + wv * wv
                else:
                    acc1 = acc1 + wv
                    sq1 = sq1 + wv * wv
            acc = acc0 + acc1
            sq = sq0 + sq1
            out_v[128 * (j & 3) + (j >> 2), :] = (acc * acc - sq) * 0.5
            return ()

        lax.fori_loop(0, CHUNK, body, (), unroll=2)

    nchunk = BPW // CHUNK
    fire(0, 0)
    for ci in range(nchunk):
        slot = ci % 2
        drain(ci * CHUNK, slot)
        if ci + 1 < nchunk:
            fire((ci + 1) * CHUNK, 1 - slot)
        compute(ci * CHUNK, slot)

    bq = (wid // 4) * 2048 + (wid % 4) * 128
    for k in range(4):
        pltpu.sync_copy(out_v.at[pl.ds(128 * k, 128), :],
                        out_hbm.at[pl.ds(bq + 512 * k, 128), :])


def _bi_interaction_sc(idx_packed, val_packed, table_lin):
    mesh = plsc.VectorSubcoreMesh(core_axis_name="c", subcore_axis_name="s")
    fn = pl.kernel(
        _sc_body,
        out_type=jax.ShapeDtypeStruct((B, E), jnp.float32),
        mesh=mesh,
        compiler_params=pltpu.CompilerParams(
            use_tc_tiling_on_sc=False, needs_layout_passes=False),
        scratch_types=[
            pltpu.VMEM((BPW, FP), jnp.int32),
            pltpu.VMEM((BPW, FP), jnp.float32),
            pltpu.VMEM((RPC, E), jnp.float32),
            pltpu.VMEM((RPC, E), jnp.float32),
            pltpu.VMEM((BPW, E), jnp.float32),
            pltpu.SemaphoreType.DMA,
            pltpu.SemaphoreType.DMA,
        ],
    )
    return fn(table_lin, idx_packed, val_packed)


def _mlp_body(bi_ref, w1_ref, b1_ref, w2_ref, b2_ref, wo_ref, bo_ref,
              out_ref):
    x = bi_ref[...]
    h = jnp.dot(x, w1_ref[...], preferred_element_type=jnp.float32)
    h = jnp.maximum(h + b1_ref[...], 0.0)
    h = jnp.dot(h, w2_ref[...], preferred_element_type=jnp.float32)
    h = jnp.maximum(h + b2_ref[...], 0.0)
    o = jnp.dot(h, wo_ref[...], preferred_element_type=jnp.float32)
    o = o + bo_ref[...]
    out_ref[...] = 1.0 / (1.0 + jnp.exp(-o))


def _mlp_tc(bi, W1, b1, W2, b2, Wo, bo):
    nblk = 8
    rows = B // 8
    blk = rows // nblk
    eye8 = jnp.eye(8, dtype=jnp.float32)
    w1b = jnp.kron(eye8, W1)
    b1b = jnp.tile(b1, 8).reshape(1, 256)
    w2b = jnp.kron(eye8, W2)
    b2b = jnp.tile(b2, 8).reshape(1, 256)
    wob = jnp.kron(eye8, Wo)
    out = pl.pallas_call(
        _mlp_body,
        grid=(nblk,),
        in_specs=[
            pl.BlockSpec((blk, 128), lambda i: (i, 0)),
            pl.BlockSpec((128, 256), lambda i: (0, 0)),
            pl.BlockSpec((1, 256), lambda i: (0, 0)),
            pl.BlockSpec((256, 256), lambda i: (0, 0)),
            pl.BlockSpec((1, 256), lambda i: (0, 0)),
            pl.BlockSpec((256, 8), lambda i: (0, 0)),
            pl.BlockSpec((1, 1), lambda i: (0, 0)),
        ],
        out_specs=pl.BlockSpec((blk, 8), lambda i: (i, 0)),
        out_shape=jax.ShapeDtypeStruct((rows, 8), jnp.float32),
    )(bi.reshape(rows, 128), w1b, b1b, w2b, b2b, wob, bo.reshape(1, 1))
    return out.reshape(B, 1)


@jax.jit
def _nfm(feat_index, feat_value, emb_table, W1, b1, W2, b2, Wo, bo):
    table_lin = _linearize_table_tc(emb_table)
    idx_packed, val_packed = _pack_inputs_tc(feat_index, feat_value)
    bi = _bi_interaction_sc(idx_packed.reshape(B, FP),
                            val_packed.reshape(B, FP), table_lin)
    return _mlp_tc(bi, W1, b1, W2, b2, Wo, bo)


def kernel(feat_index, feat_value, emb_table, W1, b1, W2, b2, Wo, bo):
    return _nfm(feat_index, feat_value, emb_table, W1, b1, W2, b2, Wo, bo)


# speedup vs baseline: 3.8684x; 1.0029x over previous
import functools

import numpy as np

import jax
import jax.numpy as jnp
from jax import lax
from jax.experimental import pallas as pl
from jax.experimental.pallas import tpu as pltpu
from jax.experimental.pallas import tpu_sc as plsc

B = 16384
F = 26
FP = 32
E = 16
NC = 2
NS = 16
NW = NC * NS
BPW = B // NW
CHUNK = 64
RPC = CHUNK * FP

RL_CH = 16384
RL_GRID = 62
RL_SLAB = RL_CH // 8
RL_SHIFT = RL_SLAB.bit_length() - 1

PK_BLK = 2048
PK_GRID = B // PK_BLK
PK_SLAB = PK_BLK // 4


def _relayout_body(in_ref, q_ref, out_ref):
    x = in_ref[...]
    ch = x.shape[1]
    x2 = x.reshape(128, ch // 8)
    out_ref[...] = lax.dot_general(
        x2, q_ref[...], (((0,), (0,)), ((), ())),
        preferred_element_type=jnp.float32)


def _perm_q():
    q = np.zeros((128, 128), np.float32)
    for d in range(E):
        for j in range(8):
            q[8 * d + j, E * j + d] = 1.0
    return jnp.asarray(q)


def _linearize_table_tc(emb_table):
    tt = jnp.transpose(emb_table)
    lin = pl.pallas_call(
        _relayout_body,
        grid=(RL_GRID,),
        in_specs=[pl.BlockSpec((E, RL_CH), lambda i: (0, i)),
                  pl.BlockSpec((128, 128), lambda i: (0, 0))],
        out_specs=pl.BlockSpec((RL_CH // 8, 128), lambda i: (i, 0)),
        out_shape=jax.ShapeDtypeStruct((RL_GRID * RL_CH // 8, 128),
                                       jnp.float32),
    )(tt, _perm_q())
    return lin.reshape(RL_GRID * RL_CH, E)


def _pack_body(idx_ref, val_ref, p_ref, iout_ref, vout_ref):
    xi = idx_ref[...]
    vidx = ((xi & ~(RL_CH - 1)) | ((xi & (RL_SLAB - 1)) << 3)
            | ((xi >> RL_SHIFT) & 7)).astype(jnp.float32)
    xv = val_ref[...]
    iacc = None
    vacc = None
    for k in range(4):
        p_k = p_ref[k]
        sl = slice(PK_SLAB * k, PK_SLAB * (k + 1))
        it = lax.dot_general(vidx[:, sl], p_k, (((0,), (0,)), ((), ())),
                             preferred_element_type=jnp.float32)
        vt = lax.dot_general(xv[:, sl], p_k, (((0,), (0,)), ((), ())),
                             preferred_element_type=jnp.float32)
        iacc = it if iacc is None else iacc + it
        vacc = vt if vacc is None else vacc + vt
    lane = lax.broadcasted_iota(jnp.int32, (PK_SLAB, 128), 1)
    rowi = lax.broadcasted_iota(jnp.int32, (PK_SLAB, 128), 0)
    pad = (lane % FP) >= F
    iout_ref[...] = jnp.where(pad, (rowi * 8 + (lane & 7)) & (RL_CH - 1),
                              iacc.astype(jnp.int32))
    vout_ref[...] = vacc


def _pack_p():
    p = np.zeros((4, F, 128), np.float32)
    for k in range(4):
        for g in range(F):
            p[k, g, FP * k + g] = 1.0
    return jnp.asarray(p)


def _pack_inputs_tc(feat_index, feat_value):
    return pl.pallas_call(
        _pack_body,
        grid=(PK_GRID,),
        in_specs=[pl.BlockSpec((F, PK_BLK), lambda i: (0, i)),
                  pl.BlockSpec((F, PK_BLK), lambda i: (0, i)),
                  pl.BlockSpec((4, F, 128), lambda i: (0, 0, 0))],
        out_specs=[pl.BlockSpec((PK_SLAB, 128), lambda i: (i, 0)),
                   pl.BlockSpec((PK_SLAB, 128), lambda i: (i, 0))],
        out_shape=[jax.ShapeDtypeStruct((B // 4, 128), jnp.int32),
                   jax.ShapeDtypeStruct((B // 4, 128), jnp.float32)],
    )(jnp.transpose(feat_index.astype(jnp.int32)),
      jnp.transpose(feat_value), _pack_p())


def _sc_body(table, idx_hbm, val_hbm, out_hbm,
             idx_v, val_v, rows_a, rows_b, out_v, sem_a, sem_b):
    c = lax.axis_index("c")
    s = lax.axis_index("s")
    wid = s * NC + c

    pltpu.sync_copy(idx_hbm.at[pl.ds(wid * BPW // 4, BPW // 4), :], idx_v)
    pltpu.sync_copy(val_hbm.at[pl.ds(wid * BPW, BPW), :], val_v)

    bufs = (rows_a, rows_b)
    sems = (sem_a, sem_b)
    GPC = CHUNK // 4

    def gather_desc(jbase, slot, i):
        return pltpu.make_async_copy(
            table.at[idx_v.at[jbase // 4 + i]],
            bufs[slot].at[pl.ds(i * 128, 128), :],
            sems[slot])

    def fire(jbase, slot):
        def body(i, _):
            gather_desc(jbase, slot, i).start()
            return ()
        lax.fori_loop(0, GPC, body, ())

    def drain(jbase, slot):
        def body(i, _):
            gather_desc(jbase, slot, i).wait()
            return ()
        lax.fori_loop(0, GPC, body, ())

    def compute(jbase, slot):
        rows = bufs[slot]

        def body(i, _):
            j = jbase + i
            r0 = i * FP
            v_lo = val_v[j, pl.ds(0, E)]
            v_hi = val_v[j, pl.ds(E, E)]
            acc0 = jnp.zeros((E,), jnp.float32)
            acc1 = jnp.zeros((E,), jnp.float32)
            sq0 = jnp.zeros((E,), jnp.float32)
            sq1 = jnp.zeros((E,), jnp.float32)
            for f in range(F):
                row = rows[r0 + f, :]
                scalar = v_lo[f] if f < E else v_hi[f - E]
                wv = row * jnp.broadcast_to(scalar, (E,))
                if f % 2 == 0:
                    acc0 = acc0 + wv
                    sq0 = sq0 + wv * wv
                else:
                    acc1 = acc1 + wv
                    sq1 = sq1 + wv * wv
            acc = acc0 + acc1
            sq = sq0 + sq1
            out_v[128 * (j & 3) + (j >> 2), :] = (acc * acc - sq) * 0.5
            return ()

        lax.fori_loop(0, CHUNK, body, (), unroll=2)

    nchunk = BPW // CHUNK
    fire(0, 0)
    for ci in range(nchunk):
        slot = ci % 2
        drain(ci * CHUNK, slot)
        if ci + 1 < nchunk:
            fire((ci + 1) * CHUNK, 1 - slot)
        compute(ci * CHUNK, slot)

    bq = (wid // 4) * 2048 + (wid % 4) * 128
    for k in range(4):
        pltpu.sync_copy(out_v.at[pl.ds(128 * k, 128), :],
                        out_hbm.at[pl.ds(bq + 512 * k, 128), :])


def _bi_interaction_sc(idx_packed, val_packed, table_lin):
    mesh = plsc.VectorSubcoreMesh(core_axis_name="c", subcore_axis_name="s")
    fn = pl.kernel(
        _sc_body,
        out_type=jax.ShapeDtypeStruct((B, E), jnp.float32),
        mesh=mesh,
        compiler_params=pltpu.CompilerParams(
            use_tc_tiling_on_sc=False, needs_layout_passes=False),
        scratch_types=[
            pltpu.VMEM((BPW // 4, 128), jnp.int32),
            pltpu.VMEM((BPW, FP), jnp.float32),
            pltpu.VMEM((RPC, E), jnp.float32),
            pltpu.VMEM((RPC, E), jnp.float32),
            pltpu.VMEM((BPW, E), jnp.float32),
            pltpu.SemaphoreType.DMA,
            pltpu.SemaphoreType.DMA,
        ],
    )
    return fn(table_lin, idx_packed, val_packed)


def _mlp_body(bi_ref, w1_ref, b1_ref, w2_ref, b2_ref, wo_ref, bo_ref,
              out_ref):
    x = bi_ref[...]
    h = jnp.dot(x, w1_ref[...], preferred_element_type=jnp.float32)
    h = jnp.maximum(h + b1_ref[...], 0.0)
    h = jnp.dot(h, w2_ref[...], preferred_element_type=jnp.float32)
    h = jnp.maximum(h + b2_ref[...], 0.0)
    o = jnp.dot(h, wo_ref[...], preferred_element_type=jnp.float32)
    o = o + bo_ref[...]
    out_ref[...] = 1.0 / (1.0 + jnp.exp(-o))


def _mlp_tc(bi, W1, b1, W2, b2, Wo, bo):
    nblk = 8
    rows = B // 8
    blk = rows // nblk
    eye8 = jnp.eye(8, dtype=jnp.float32)
    w1b = jnp.kron(eye8, W1)
    b1b = jnp.tile(b1, 8).reshape(1, 256)
    w2b = jnp.kron(eye8, W2)
    b2b = jnp.tile(b2, 8).reshape(1, 256)
    wob = jnp.kron(eye8, Wo)
    out = pl.pallas_call(
        _mlp_body,
        grid=(nblk,),
        in_specs=[
            pl.BlockSpec((blk, 128), lambda i: (i, 0)),
            pl.BlockSpec((128, 256), lambda i: (0, 0)),
            pl.BlockSpec((1, 256), lambda i: (0, 0)),
            pl.BlockSpec((256, 256), lambda i: (0, 0)),
            pl.BlockSpec((1, 256), lambda i: (0, 0)),
            pl.BlockSpec((256, 8), lambda i: (0, 0)),
            pl.BlockSpec((1, 1), lambda i: (0, 0)),
        ],
        out_specs=pl.BlockSpec((blk, 8), lambda i: (i, 0)),
        out_shape=jax.ShapeDtypeStruct((rows, 8), jnp.float32),
    )(bi.reshape(rows, 128), w1b, b1b, w2b, b2b, wob, bo.reshape(1, 1))
    return out.reshape(B, 1)


@jax.jit
def _nfm(feat_index, feat_value, emb_table, W1, b1, W2, b2, Wo, bo):
    table_lin = _linearize_table_tc(emb_table)
    idx_packed, val_packed = _pack_inputs_tc(feat_index, feat_value)
    bi = _bi_interaction_sc(idx_packed, val_packed.reshape(B, FP),
                            table_lin)
    return _mlp_tc(bi, W1, b1, W2, b2, Wo, bo)


def kernel(feat_index, feat_value, emb_table, W1, b1, W2, b2, Wo, bo):
    return _nfm(feat_index, feat_value, emb_table, W1, b1, W2, b2, Wo, bo)


# speedup vs baseline: 3.8687x; 1.0001x over previous
import functools

import numpy as np

import jax
import jax.numpy as jnp
from jax import lax
from jax.experimental import pallas as pl
from jax.experimental.pallas import tpu as pltpu
from jax.experimental.pallas import tpu_sc as plsc

B = 16384
F = 26
FP = 32
E = 16
NC = 2
NS = 16
NW = NC * NS
BPW = B // NW
CHUNK = 64
RPC = CHUNK * FP

RL_CH = 16384
RL_GRID = 62
RL_SLAB = RL_CH // 8
RL_SHIFT = RL_SLAB.bit_length() - 1

PK_BLK = 2048
PK_GRID = B // PK_BLK
PK_SLAB = PK_BLK // 4


def _relayout_body(in_ref, q_ref, out_ref):
    x = in_ref[...]
    ch = x.shape[1]
    x2 = x.reshape(128, ch // 8)
    out_ref[...] = lax.dot_general(
        x2, q_ref[...], (((0,), (0,)), ((), ())),
        preferred_element_type=jnp.float32)


def _perm_q():
    q = np.zeros((128, 128), np.float32)
    for d in range(E):
        for j in range(8):
            q[8 * d + j, E * j + d] = 1.0
    return jnp.asarray(q)


def _linearize_table_tc(emb_table):
    tt = jnp.transpose(emb_table)
    lin = pl.pallas_call(
        _relayout_body,
        grid=(RL_GRID,),
        in_specs=[pl.BlockSpec((E, RL_CH), lambda i: (0, i)),
                  pl.BlockSpec((128, 128), lambda i: (0, 0))],
        out_specs=pl.BlockSpec((RL_CH // 8, 128), lambda i: (i, 0)),
        out_shape=jax.ShapeDtypeStruct((RL_GRID * RL_CH // 8, 128),
                                       jnp.float32),
    )(tt, _perm_q())
    return lin.reshape(RL_GRID * RL_CH, E)


def _pack_body(idx_ref, val_ref, p_ref, iout_ref, vout_ref):
    xi = idx_ref[...]
    vidx = ((xi & ~(RL_CH - 1)) | ((xi & (RL_SLAB - 1)) << 3)
            | ((xi >> RL_SHIFT) & 7)).astype(jnp.float32)
    xv = val_ref[...]
    iacc = None
    vacc = None
    for k in range(4):
        p_k = p_ref[k]
        sl = slice(PK_SLAB * k, PK_SLAB * (k + 1))
        it = lax.dot_general(vidx[:, sl], p_k, (((0,), (0,)), ((), ())),
                             preferred_element_type=jnp.float32)
        vt = lax.dot_general(xv[:, sl], p_k, (((0,), (0,)), ((), ())),
                             preferred_element_type=jnp.float32)
        iacc = it if iacc is None else iacc + it
        vacc = vt if vacc is None else vacc + vt
    lane = lax.broadcasted_iota(jnp.int32, (PK_SLAB, 128), 1)
    rowi = lax.broadcasted_iota(jnp.int32, (PK_SLAB, 128), 0)
    pad = (lane % FP) >= F
    iout_ref[...] = jnp.where(pad, (rowi * 8 + (lane & 7)) & (RL_CH - 1),
                              iacc.astype(jnp.int32))
    vout_ref[...] = vacc


def _pack_p():
    p = np.zeros((4, F, 128), np.float32)
    for k in range(4):
        for g in range(F):
            p[k, g, FP * k + g] = 1.0
    return jnp.asarray(p)


def _pack_inputs_tc(feat_index, feat_value):
    return pl.pallas_call(
        _pack_body,
        grid=(PK_GRID,),
        in_specs=[pl.BlockSpec((F, PK_BLK), lambda i: (0, i)),
                  pl.BlockSpec((F, PK_BLK), lambda i: (0, i)),
                  pl.BlockSpec((4, F, 128), lambda i: (0, 0, 0))],
        out_specs=[pl.BlockSpec((PK_SLAB, 128), lambda i: (i, 0)),
                   pl.BlockSpec((PK_SLAB, 128), lambda i: (i, 0))],
        out_shape=[jax.ShapeDtypeStruct((B // 4, 128), jnp.int32),
                   jax.ShapeDtypeStruct((B // 4, 128), jnp.float32)],
    )(jnp.transpose(feat_index.astype(jnp.int32)),
      jnp.transpose(feat_value), _pack_p())


def _sc_body(table, idx_hbm, val_hbm, out_hbm,
             idx_v, val_v, rows_a, rows_b, out_v, sem_a, sem_b):
    c = lax.axis_index("c")
    s = lax.axis_index("s")
    wid = s * NC + c

    pltpu.sync_copy(idx_hbm.at[pl.ds(wid * BPW // 4, BPW // 4), :], idx_v)
    pltpu.sync_copy(val_hbm.at[pl.ds(wid * BPW, BPW), :], val_v)

    bufs = (rows_a, rows_b)
    sems = (sem_a, sem_b)
    GPC = CHUNK // 4

    def gather_desc(jbase, slot, i):
        return pltpu.make_async_copy(
            table.at[idx_v.at[jbase // 4 + i]],
            bufs[slot].at[pl.ds(i * 128, 128), :],
            sems[slot])

    def fire(jbase, slot):
        def body(i, _):
            gather_desc(jbase, slot, i).start()
            return ()
        lax.fori_loop(0, GPC, body, ())

    def drain(jbase, slot):
        def body(i, _):
            gather_desc(jbase, slot, i).wait()
            return ()
        lax.fori_loop(0, GPC, body, ())

    def compute(jbase, slot):
        rows = bufs[slot]

        lane_consts = [jnp.full((E,), f % E, jnp.int32) for f in range(E + 1)]

        def body(i, _):
            j = jbase + i
            r0 = i * FP
            v_lo = val_v[j, pl.ds(0, E)]
            v_hi = val_v[j, pl.ds(E, E)]
            acc0 = jnp.zeros((E,), jnp.float32)
            acc1 = jnp.zeros((E,), jnp.float32)
            sq0 = jnp.zeros((E,), jnp.float32)
            sq1 = jnp.zeros((E,), jnp.float32)
            for f in range(F):
                row = rows[r0 + f, :]
                src = v_lo if f < E else v_hi
                wv = row * jnp.take(src, lane_consts[f % E])
                if f % 2 == 0:
                    acc0 = acc0 + wv
                    sq0 = sq0 + wv * wv
                else:
                    acc1 = acc1 + wv
                    sq1 = sq1 + wv * wv
            acc = acc0 + acc1
            sq = sq0 + sq1
            out_v[128 * (j & 3) + (j >> 2), :] = (acc * acc - sq) * 0.5
            return ()

        lax.fori_loop(0, CHUNK, body, (), unroll=2)

    nchunk = BPW // CHUNK
    fire(0, 0)
    for ci in range(nchunk):
        slot = ci % 2
        drain(ci * CHUNK, slot)
        if ci + 1 < nchunk:
            fire((ci + 1) * CHUNK, 1 - slot)
        compute(ci * CHUNK, slot)

    bq = (wid // 4) * 2048 + (wid % 4) * 128
    for k in range(4):
        pltpu.sync_copy(out_v.at[pl.ds(128 * k, 128), :],
                        out_hbm.at[pl.ds(bq + 512 * k, 128), :])


def _bi_interaction_sc(idx_packed, val_packed, table_lin):
    mesh = plsc.VectorSubcoreMesh(core_axis_name="c", subcore_axis_name="s")
    fn = pl.kernel(
        _sc_body,
        out_type=jax.ShapeDtypeStruct((B, E), jnp.float32),
        mesh=mesh,
        compiler_params=pltpu.CompilerParams(
            use_tc_tiling_on_sc=False, needs_layout_passes=False),
        scratch_types=[
            pltpu.VMEM((BPW // 4, 128), jnp.int32),
            pltpu.VMEM((BPW, FP), jnp.float32),
            pltpu.VMEM((RPC, E), jnp.float32),
            pltpu.VMEM((RPC, E), jnp.float32),
            pltpu.VMEM((BPW, E), jnp.float32),
            pltpu.SemaphoreType.DMA,
            pltpu.SemaphoreType.DMA,
        ],
    )
    return fn(table_lin, idx_packed, val_packed)


def _mlp_body(bi_ref, w1_ref, b1_ref, w2_ref, b2_ref, wo_ref, bo_ref,
              out_ref):
    x = bi_ref[...]
    h = jnp.dot(x, w1_ref[...], preferred_element_type=jnp.float32)
    h = jnp.maximum(h + b1_ref[...], 0.0)
    h = jnp.dot(h, w2_ref[...], preferred_element_type=jnp.float32)
    h = jnp.maximum(h + b2_ref[...], 0.0)
    o = jnp.dot(h, wo_ref[...], preferred_element_type=jnp.float32)
    o = o + bo_ref[...]
    out_ref[...] = 1.0 / (1.0 + jnp.exp(-o))


def _mlp_tc(bi, W1, b1, W2, b2, Wo, bo):
    nblk = 8
    rows = B // 8
    blk = rows // nblk
    eye8 = jnp.eye(8, dtype=jnp.float32)
    w1b = jnp.kron(eye8, W1)
    b1b = jnp.tile(b1, 8).reshape(1, 256)
    w2b = jnp.kron(eye8, W2)
    b2b = jnp.tile(b2, 8).reshape(1, 256)
    wob = jnp.kron(eye8, Wo)
    out = pl.pallas_call(
        _mlp_body,
        grid=(nblk,),
        in_specs=[
            pl.BlockSpec((blk, 128), lambda i: (i, 0)),
            pl.BlockSpec((128, 256), lambda i: (0, 0)),
            pl.BlockSpec((1, 256), lambda i: (0, 0)),
            pl.BlockSpec((256, 256), lambda i: (0, 0)),
            pl.BlockSpec((1, 256), lambda i: (0, 0)),
            pl.BlockSpec((256, 8), lambda i: (0, 0)),
            pl.BlockSpec((1, 1), lambda i: (0, 0)),
        ],
        out_specs=pl.BlockSpec((blk, 8), lambda i: (i, 0)),
        out_shape=jax.ShapeDtypeStruct((rows, 8), jnp.float32),
    )(bi.reshape(rows, 128), w1b, b1b, w2b, b2b, wob, bo.reshape(1, 1))
    return out.reshape(B, 1)


@jax.jit
def _nfm(feat_index, feat_value, emb_table, W1, b1, W2, b2, Wo, bo):
    table_lin = _linearize_table_tc(emb_table)
    idx_packed, val_packed = _pack_inputs_tc(feat_index, feat_value)
    bi = _bi_interaction_sc(idx_packed, val_packed.reshape(B, FP),
                            table_lin)
    return _mlp_tc(bi, W1, b1, W2, b2, Wo, bo)


def kernel(feat_index, feat_value, emb_table, W1, b1, W2, b2, Wo, bo):
    return _nfm(feat_index, feat_value, emb_table, W1, b1, W2, b2, Wo, bo)


# speedup vs baseline: 3.8833x; 1.0038x over previous
import numpy as np

import jax
import jax.numpy as jnp
from jax import lax
from jax.experimental import pallas as pl
from jax.experimental.pallas import tpu as pltpu
from jax.experimental.pallas import tpu_sc as plsc

B = 16384
F = 26
FP = 32
E = 16
NC = 2
NS = 16
NW = NC * NS
BPW = B // NW
CHUNK = 64
RPC = CHUNK * FP

RL_CH = 16384
RL_GRID = 62
RL_SLAB = RL_CH // 8
RL_SHIFT = RL_SLAB.bit_length() - 1

PK_BLK = 2048
PK_GRID = B // PK_BLK
PK_SLAB = PK_BLK // 4


def _relayout_body(in_ref, q_ref, out_ref):
    x = in_ref[...]
    ch = x.shape[1]
    x2 = x.reshape(128, ch // 8)
    out_ref[...] = lax.dot_general(
        x2, q_ref[...], (((0,), (0,)), ((), ())),
        preferred_element_type=jnp.float32)


def _perm_q():
    q = np.zeros((128, 128), np.float32)
    for d in range(E):
        for j in range(8):
            q[8 * d + j, E * j + d] = 1.0
    return jnp.asarray(q)


def _linearize_table_tc(emb_table):
    tt = jnp.transpose(emb_table)
    lin = pl.pallas_call(
        _relayout_body,
        grid=(RL_GRID,),
        in_specs=[pl.BlockSpec((E, RL_CH), lambda i: (0, i)),
                  pl.BlockSpec((128, 128), lambda i: (0, 0))],
        out_specs=pl.BlockSpec((RL_CH // 8, 128), lambda i: (i, 0)),
        out_shape=jax.ShapeDtypeStruct((RL_GRID * RL_CH // 8, 128),
                                       jnp.float32),
    )(tt, _perm_q())
    return lin.reshape(RL_GRID * RL_CH, E)


def _pack_body(idx_ref, val_ref, p_ref, iout_ref, vout_ref):
    xi = idx_ref[...]
    vidx = ((xi & ~(RL_CH - 1)) | ((xi & (RL_SLAB - 1)) << 3)
            | ((xi >> RL_SHIFT) & 7)).astype(jnp.float32)
    xv = val_ref[...]
    iacc = None
    vacc = None
    for k in range(4):
        p_k = p_ref[k]
        sl = slice(PK_SLAB * k, PK_SLAB * (k + 1))
        it = lax.dot_general(vidx[:, sl], p_k, (((0,), (0,)), ((), ())),
                             preferred_element_type=jnp.float32)
        vt = lax.dot_general(xv[:, sl], p_k, (((0,), (0,)), ((), ())),
                             preferred_element_type=jnp.float32)
        iacc = it if iacc is None else iacc + it
        vacc = vt if vacc is None else vacc + vt
    lane = lax.broadcasted_iota(jnp.int32, (PK_SLAB, 128), 1)
    rowi = lax.broadcasted_iota(jnp.int32, (PK_SLAB, 128), 0)
    pad = (lane % FP) >= F
    iout_ref[...] = jnp.where(pad, (rowi * 8 + (lane & 7)) & (RL_CH - 1),
                              iacc.astype(jnp.int32))
    vout_ref[...] = vacc


def _pack_p():
    p = np.zeros((4, F, 128), np.float32)
    for k in range(4):
        for g in range(F):
            p[k, g, FP * k + g] = 1.0
    return jnp.asarray(p)


def _pack_inputs_tc(feat_index, feat_value):
    return pl.pallas_call(
        _pack_body,
        grid=(PK_GRID,),
        in_specs=[pl.BlockSpec((F, PK_BLK), lambda i: (0, i)),
                  pl.BlockSpec((F, PK_BLK), lambda i: (0, i)),
                  pl.BlockSpec((4, F, 128), lambda i: (0, 0, 0))],
        out_specs=[pl.BlockSpec((PK_SLAB, 128), lambda i: (i, 0)),
                   pl.BlockSpec((PK_SLAB, 128), lambda i: (i, 0))],
        out_shape=[jax.ShapeDtypeStruct((B // 4, 128), jnp.int32),
                   jax.ShapeDtypeStruct((B // 4, 128), jnp.float32)],
    )(jnp.transpose(feat_index.astype(jnp.int32)),
      jnp.transpose(feat_value), _pack_p())


def _sc_body(table, idx_hbm, val_hbm, out_hbm,
             idx_v, val_v, rows_a, rows_b, out_v, sem_a, sem_b):
    c = lax.axis_index("c")
    s = lax.axis_index("s")
    wid = s * NC + c

    pltpu.sync_copy(idx_hbm.at[pl.ds(wid * BPW // 4, BPW // 4), :], idx_v)
    pltpu.sync_copy(val_hbm.at[pl.ds(wid * BPW, BPW), :], val_v)

    bufs = (rows_a, rows_b)
    sems = (sem_a, sem_b)
    GPC = CHUNK // 4

    def gather_desc(jbase, slot, i):
        return pltpu.make_async_copy(
            table.at[idx_v.at[jbase // 4 + i]],
            bufs[slot].at[pl.ds(i * 128, 128), :],
            sems[slot])

    def fire(jbase, slot):
        def body(i, _):
            gather_desc(jbase, slot, i).start()
            return ()
        lax.fori_loop(0, GPC, body, ())

    def drain(jbase, slot):
        def body(i, _):
            gather_desc(jbase, slot, i).wait()
            return ()
        lax.fori_loop(0, GPC, body, ())

    def compute(jbase, slot):
        rows = bufs[slot]

        lane_consts = [jnp.full((E,), f % E, jnp.int32) for f in range(E + 1)]

        def body(i, _):
            j = jbase + i
            r0 = i * FP
            v_lo = val_v[j, pl.ds(0, E)]
            v_hi = val_v[j, pl.ds(E, E)]
            acc0 = jnp.zeros((E,), jnp.float32)
            acc1 = jnp.zeros((E,), jnp.float32)
            sq0 = jnp.zeros((E,), jnp.float32)
            sq1 = jnp.zeros((E,), jnp.float32)
            for f in range(F):
                row = rows[r0 + f, :]
                src = v_lo if f < E else v_hi
                wv = row * jnp.take(src, lane_consts[f % E])
                if f % 2 == 0:
                    acc0 = acc0 + wv
                    sq0 = sq0 + wv * wv
                else:
                    acc1 = acc1 + wv
                    sq1 = sq1 + wv * wv
            acc = acc0 + acc1
            sq = sq0 + sq1
            out_v[128 * (j & 3) + (j >> 2), :] = (acc * acc - sq) * 0.5
            return ()

        lax.fori_loop(0, CHUNK, body, (), unroll=2)

    nchunk = BPW // CHUNK
    fire(0, 0)
    for ci in range(nchunk):
        slot = ci % 2
        drain(ci * CHUNK, slot)
        if ci + 1 < nchunk:
            fire((ci + 1) * CHUNK, 1 - slot)
        compute(ci * CHUNK, slot)

    bq = (wid // 4) * 2048 + (wid % 4) * 128
    for k in range(4):
        pltpu.sync_copy(out_v.at[pl.ds(128 * k, 128), :],
                        out_hbm.at[pl.ds(bq + 512 * k, 128), :])


def _bi_interaction_sc(idx_packed, val_packed, table_lin):
    mesh = plsc.VectorSubcoreMesh(core_axis_name="c", subcore_axis_name="s")
    fn = pl.kernel(
        _sc_body,
        out_type=jax.ShapeDtypeStruct((B, E), jnp.float32),
        mesh=mesh,
        compiler_params=pltpu.CompilerParams(
            use_tc_tiling_on_sc=False, needs_layout_passes=False),
        scratch_types=[
            pltpu.VMEM((BPW // 4, 128), jnp.int32),
            pltpu.VMEM((BPW, FP), jnp.float32),
            pltpu.VMEM((RPC, E), jnp.float32),
            pltpu.VMEM((RPC, E), jnp.float32),
            pltpu.VMEM((BPW, E), jnp.float32),
            pltpu.SemaphoreType.DMA,
            pltpu.SemaphoreType.DMA,
        ],
    )
    return fn(table_lin, idx_packed, val_packed)


def _mlp_body(bi_ref, w1_ref, b1_ref, w2_ref, b2_ref, wo_ref, bo_ref,
              out_ref):
    x = bi_ref[...]
    h = jnp.dot(x, w1_ref[...], preferred_element_type=jnp.float32)
    h = jnp.maximum(h + b1_ref[...], 0.0)
    h = jnp.dot(h, w2_ref[...], preferred_element_type=jnp.float32)
    h = jnp.maximum(h + b2_ref[...], 0.0)
    o = jnp.dot(h, wo_ref[...], preferred_element_type=jnp.float32)
    o = o + bo_ref[...]
    out_ref[...] = 1.0 / (1.0 + jnp.exp(-o))


def _mlp_tc(bi, W1, b1, W2, b2, Wo, bo):
    nblk = 8
    rows = B // 8
    blk = rows // nblk
    eye8 = jnp.eye(8, dtype=jnp.float32)
    w1b = jnp.kron(eye8, W1)
    b1b = jnp.tile(b1, 8).reshape(1, 256)
    w2b = jnp.kron(eye8, W2)
    b2b = jnp.tile(b2, 8).reshape(1, 256)
    wob = jnp.kron(eye8, Wo)
    out = pl.pallas_call(
        _mlp_body,
        grid=(nblk,),
        in_specs=[
            pl.BlockSpec((blk, 128), lambda i: (i, 0)),
            pl.BlockSpec((128, 256), lambda i: (0, 0)),
            pl.BlockSpec((1, 256), lambda i: (0, 0)),
            pl.BlockSpec((256, 256), lambda i: (0, 0)),
            pl.BlockSpec((1, 256), lambda i: (0, 0)),
            pl.BlockSpec((256, 8), lambda i: (0, 0)),
            pl.BlockSpec((1, 1), lambda i: (0, 0)),
        ],
        out_specs=pl.BlockSpec((blk, 8), lambda i: (i, 0)),
        out_shape=jax.ShapeDtypeStruct((rows, 8), jnp.float32),
    )(bi.reshape(rows, 128), w1b, b1b, w2b, b2b, wob, bo.reshape(1, 1))
    return out.reshape(B, 1)


@jax.jit
def _nfm(feat_index, feat_value, emb_table, W1, b1, W2, b2, Wo, bo):
    table_lin = _linearize_table_tc(emb_table)
    idx_packed, val_packed = _pack_inputs_tc(feat_index, feat_value)
    bi = _bi_interaction_sc(idx_packed, val_packed.reshape(B, FP),
                            table_lin)
    return _mlp_tc(bi, W1, b1, W2, b2, Wo, bo)


def kernel(feat_index, feat_value, emb_table, W1, b1, W2, b2, Wo, bo):
    return _nfm(feat_index, feat_value, emb_table, W1, b1, W2, b2, Wo, bo)
